# baseline (speedup 1.0000x reference)
import functools

import jax
import jax.numpy as jnp
from jax import lax
from jax.experimental import pallas as pl
from jax.experimental.pallas import tpu as pltpu
from jax.experimental.pallas import tpu_sc as plsc

U = 50000
D = 32
E = 1600000
NT = 16
CHUNK = 512
KB = CHUNK // 128
EPT = -(-E // (NT * CHUNK)) * CHUNK
EPAD = EPT * NT
NCH = EPT // CHUNK
WBR = 80
NBLK = U // WBR
BPT = -(-NBLK // NT)


def _body(final, *refs):
    if final:
        (uu_p, ui_p, uiT_p, ii_p,
         tab_u, tab_i, ue0, ie0, out_u, out_i,
         acc, comb, rows, wb, b0, b1, gsem, ssem) = refs
    else:
        (uu_p, ui_p, uiT_p, ii_p,
         tab_u, tab_i, out_u, out_i,
         acc, comb, rows, wb, b0, b1, gsem, ssem) = refs
        ue0 = ie0 = None

    cid = lax.axis_index("c")
    sid = lax.axis_index("s")
    zero = jnp.zeros((16,), jnp.float32)

    @pl.loop(0, WBR)
    def _(r):
        rows[r, 0:16] = zero
        rows[r, 16:32] = zero

    @pl.loop(0, BPT)
    def _(i):
        blk = sid + i * NT

        @pl.when(blk < NBLK)
        def _():
            pltpu.sync_copy(rows.at[pl.ds(0, WBR)],
                            acc.at[pl.ds(blk * WBR, WBR)])

    plsc.subcore_barrier()

    def do_spmm(packed, table):
        base = sid * NCH * 3

        @pl.loop(0, NCH)
        def _(ci):
            pltpu.sync_copy(packed.at[pl.ds(base + ci * 3, 3)], comb)
            pltpu.async_copy(table.at[comb.at[0]], rows, gsem).wait()

            @plsc.parallel_loop(0, CHUNK // 16)
            def _(g):
                vv = comb[2, pl.ds(g * 16, 16)].view(jnp.float32)
                if final:
                    vv = vv * 0.25
                for t in range(16):
                    v = vv[t]
                    r = g * 16 + t
                    rows[r, 0:16] = rows[r, 0:16] * v
                    rows[r, 16:32] = rows[r, 16:32] * v

            pltpu.async_copy(rows, acc.at[comb.at[1]], ssem,
                             add=True).wait()

    @pl.when(cid == 0)
    def _():
        do_spmm(uu_p, tab_u)
        do_spmm(ui_p, tab_i)

    @pl.when(cid == 1)
    def _():
        do_spmm(ii_p, tab_i)
        do_spmm(uiT_p, tab_u)

    plsc.subcore_barrier()

    def writeback(out_ref, e0_ref, a1_ref):
        @pl.loop(0, BPT)
        def _(i):
            blk = sid + i * NT

            @pl.when(blk < NBLK)
            def _():
                rr = blk * WBR
                pltpu.sync_copy(acc.at[pl.ds(rr, WBR)], wb)
                if final:
                    pltpu.sync_copy(e0_ref.at[pl.ds(rr, WBR)], b0)
                    pltpu.sync_copy(a1_ref.at[pl.ds(rr, WBR)], b1)

                    @plsc.parallel_loop(0, WBR, unroll=5)
                    def _(r):
                        for h in (0, 16):
                            s = (b0[r, h:h + 16] + 0.5 * b1[r, h:h + 16]
                                 + wb[r, h:h + 16])
                            wb[r, h:h + 16] = s * (1.0 / 3.0)
                pltpu.sync_copy(wb, out_ref.at[pl.ds(rr, WBR)])

    @pl.when(cid == 0)
    def _():
        writeback(out_u, ue0, tab_u)

    @pl.when(cid == 1)
    def _():
        writeback(out_i, ie0, tab_i)


def _build(final):
    scratch = [
        pltpu.VMEM_SHARED((U, D), jnp.float32),
        pltpu.VMEM((3, CHUNK), jnp.int32),
        pltpu.VMEM((CHUNK, D), jnp.float32),
        pltpu.VMEM((WBR, D), jnp.float32),
        pltpu.VMEM((WBR, D), jnp.float32),
        pltpu.VMEM((WBR, D), jnp.float32),
        pltpu.SemaphoreType.DMA,
        pltpu.SemaphoreType.DMA,
    ]
    mesh = plsc.VectorSubcoreMesh(core_axis_name="c", subcore_axis_name="s")
    out_type = (jax.ShapeDtypeStruct((U, D), jnp.float32),
                jax.ShapeDtypeStruct((U, D), jnp.float32))
    return pl.kernel(functools.partial(_body, final), out_type=out_type,
                     mesh=mesh, scratch_types=scratch,
                     compiler_params=pltpu.CompilerParams(
                         use_tc_tiling_on_sc=False))


_layer1 = _build(final=False)
_layer2 = _build(final=True)


@jax.jit
def _run(uu_ei, uu_v, ui_ei, ui_v, ii_ei, ii_v, ue, ie):
    pad = EPAD - E

    def prep(src_col, dst_col, v):
        s2 = jnp.pad(src_col, (0, pad)).reshape(-1, 1, CHUNK)
        d2 = jnp.pad(dst_col, (0, pad)).reshape(-1, 1, CHUNK)
        vb = lax.bitcast_convert_type(jnp.pad(v, (0, pad)),
                                      jnp.int32).reshape(-1, 1, CHUNK)
        return jnp.concatenate([s2, d2, vb], axis=1).reshape(-1, CHUNK)

    uu_p = prep(uu_ei[1], uu_ei[0], uu_v)
    ui_p = prep(ui_ei[1], ui_ei[0], ui_v)
    uiT_p = prep(ui_ei[0], ui_ei[1], ui_v)
    ii_p = prep(ii_ei[1], ii_ei[0], ii_v)
    args = (uu_p, ui_p, uiT_p, ii_p)

    acc_u, acc_i = _layer1(*args, ue, ie)
    out_u, out_i = _layer2(*args, acc_u, acc_i, ue, ie)
    return out_u, out_i


def kernel(uu_edge_index, uu_values, ui_edge_index, ui_values,
           ii_edge_index, ii_values, user_embedding, item_embedding):
    return _run(uu_edge_index, uu_values, ui_edge_index, ui_values,
                ii_edge_index, ii_values, user_embedding, item_embedding)


# speedup vs baseline: 1.1396x; 1.1396x over previous
import functools

import jax
import jax.numpy as jnp
from jax import lax
from jax.experimental import pallas as pl
from jax.experimental.pallas import tpu as pltpu
from jax.experimental.pallas import tpu_sc as plsc

U = 50000
D = 32
E = 1600000
NT = 16
CHUNK = 512
KB = CHUNK // 128
EPT = -(-E // (NT * CHUNK)) * CHUNK
EPAD = EPT * NT
NCH = EPT // CHUNK
WBR = 80
NBLK = U // WBR
BPT = -(-NBLK // NT)


def _body(final, *refs):
    if final:
        (uu_p, ui_p, uiT_p, ii_p,
         tab_u, tab_i, ue0, ie0, out_u, out_i,
         acc, comb0, comb1, rows, wb, b0, b1, gsem, ssem, isem) = refs
    else:
        (uu_p, ui_p, uiT_p, ii_p,
         tab_u, tab_i, out_u, out_i,
         acc, comb0, comb1, rows, wb, b0, b1, gsem, ssem, isem) = refs
        ue0 = ie0 = None

    cid = lax.axis_index("c")
    sid = lax.axis_index("s")
    zero = jnp.zeros((16,), jnp.float32)

    @pl.loop(0, WBR)
    def _(r):
        rows[r, 0:16] = zero
        rows[r, 16:32] = zero

    @pl.loop(0, BPT)
    def _(i):
        blk = sid + i * NT

        @pl.when(blk < NBLK)
        def _():
            pltpu.sync_copy(rows.at[pl.ds(0, WBR)],
                            acc.at[pl.ds(blk * WBR, WBR)])

    plsc.subcore_barrier()

    def do_spmm(packed, table):
        base = sid * NCH * 3
        comb = (comb0, comb1)

        def fetch(b, ci):
            pltpu.async_copy(packed.at[pl.ds(base + ci * 3, 3)], comb[b],
                             isem)

        def fetch_wait(b, ci):
            pltpu.make_async_copy(packed.at[pl.ds(base + ci * 3, 3)],
                                  comb[b], isem).wait()

        def process(b):
            pltpu.async_copy(table.at[comb[b].at[0]], rows, gsem).wait()

            @plsc.parallel_loop(0, CHUNK // 16)
            def _(g):
                vv = comb[b][2, pl.ds(g * 16, 16)].view(jnp.float32)
                if final:
                    vv = vv * 0.25
                for t in range(16):
                    v = vv[t]
                    r = g * 16 + t
                    rows[r, 0:16] = rows[r, 0:16] * v
                    rows[r, 16:32] = rows[r, 16:32] * v

            pltpu.async_copy(rows, acc.at[comb[b].at[1]], ssem,
                             add=True).wait()

        fetch(0, 0)
        fetch_wait(0, 0)

        @pl.loop(0, NCH // 2)
        def _(k):
            c0 = 2 * k
            fetch(1, c0 + 1)
            process(0)
            fetch_wait(1, c0 + 1)

            @pl.when(k + 1 < NCH // 2)
            def _():
                fetch(0, c0 + 2)

            process(1)

            @pl.when(k + 1 < NCH // 2)
            def _():
                fetch_wait(0, c0 + 2)

    @pl.when(cid == 0)
    def _():
        do_spmm(uu_p, tab_u)
        do_spmm(ui_p, tab_i)

    @pl.when(cid == 1)
    def _():
        do_spmm(ii_p, tab_i)
        do_spmm(uiT_p, tab_u)

    plsc.subcore_barrier()

    def writeback(out_ref, e0_ref, a1_ref):
        @pl.loop(0, BPT)
        def _(i):
            blk = sid + i * NT

            @pl.when(blk < NBLK)
            def _():
                rr = blk * WBR
                pltpu.sync_copy(acc.at[pl.ds(rr, WBR)], wb)
                if final:
                    pltpu.sync_copy(e0_ref.at[pl.ds(rr, WBR)], b0)
                    pltpu.sync_copy(a1_ref.at[pl.ds(rr, WBR)], b1)

                    @plsc.parallel_loop(0, WBR, unroll=5)
                    def _(r):
                        for h in (0, 16):
                            s = (b0[r, h:h + 16] + 0.5 * b1[r, h:h + 16]
                                 + wb[r, h:h + 16])
                            wb[r, h:h + 16] = s * (1.0 / 3.0)
                pltpu.sync_copy(wb, out_ref.at[pl.ds(rr, WBR)])

    @pl.when(cid == 0)
    def _():
        writeback(out_u, ue0, tab_u)

    @pl.when(cid == 1)
    def _():
        writeback(out_i, ie0, tab_i)


def _build(final):
    scratch = [
        pltpu.VMEM_SHARED((U, D), jnp.float32),
        pltpu.VMEM((3, CHUNK), jnp.int32),
        pltpu.VMEM((3, CHUNK), jnp.int32),
        pltpu.VMEM((CHUNK, D), jnp.float32),
        pltpu.VMEM((WBR, D), jnp.float32),
        pltpu.VMEM((WBR, D), jnp.float32),
        pltpu.VMEM((WBR, D), jnp.float32),
        pltpu.SemaphoreType.DMA,
        pltpu.SemaphoreType.DMA,
        pltpu.SemaphoreType.DMA,
    ]
    mesh = plsc.VectorSubcoreMesh(core_axis_name="c", subcore_axis_name="s")
    out_type = (jax.ShapeDtypeStruct((U, D), jnp.float32),
                jax.ShapeDtypeStruct((U, D), jnp.float32))
    return pl.kernel(functools.partial(_body, final), out_type=out_type,
                     mesh=mesh, scratch_types=scratch,
                     compiler_params=pltpu.CompilerParams(
                         use_tc_tiling_on_sc=False))


_layer1 = _build(final=False)
_layer2 = _build(final=True)


@jax.jit
def _run(uu_ei, uu_v, ui_ei, ui_v, ii_ei, ii_v, ue, ie):
    pad = EPAD - E

    def prep(src_col, dst_col, v):
        s2 = jnp.pad(src_col, (0, pad)).reshape(-1, 1, CHUNK)
        d2 = jnp.pad(dst_col, (0, pad)).reshape(-1, 1, CHUNK)
        vb = lax.bitcast_convert_type(jnp.pad(v, (0, pad)),
                                      jnp.int32).reshape(-1, 1, CHUNK)
        return jnp.concatenate([s2, d2, vb], axis=1).reshape(-1, CHUNK)

    uu_p = prep(uu_ei[1], uu_ei[0], uu_v)
    ui_p = prep(ui_ei[1], ui_ei[0], ui_v)
    uiT_p = prep(ui_ei[0], ui_ei[1], ui_v)
    ii_p = prep(ii_ei[1], ii_ei[0], ii_v)
    args = (uu_p, ui_p, uiT_p, ii_p)

    acc_u, acc_i = _layer1(*args, ue, ie)
    out_u, out_i = _layer2(*args, acc_u, acc_i, ue, ie)
    return out_u, out_i


def kernel(uu_edge_index, uu_values, ui_edge_index, ui_values,
           ii_edge_index, ii_values, user_embedding, item_embedding):
    return _run(uu_edge_index, uu_values, ui_edge_index, ui_values,
                ii_edge_index, ii_values, user_embedding, item_embedding)
